# Initial kernel scaffold; baseline (speedup 1.0000x reference)
import jax
import jax.numpy as jnp
from jax.experimental import pallas as pl


def kernel(query, value, reference_points, spatial_shapes, level_start_index, W_samp, b_samp, W_attn, b_attn, W_val, b_val, W_out, b_out):
    raise NotImplementedError("write your pallas kernel here")


# speedup vs baseline: 2003.4941x; 2003.4941x over previous
import functools
import jax
import jax.numpy as jnp
from jax import lax
from jax.experimental import pallas as pl
from jax.experimental.pallas import tpu as pltpu
from jax.experimental.pallas import tpu_sc as plsc

EMBED = 256
HEADS = 8
LEVELS = 4
POINTS = 4
HD = EMBED // HEADS
LP = LEVELS * POINTS
NQ = 5440
BS = 2
ROWS = BS * NQ
BLK = 640
NW = 32
RPW = ROWS // NW
CH = 4


def _tc_pre_body(q_ref, v_ref, rx_ref, ry_ref, boff_ref,
                 wx_ref, bx_ref, wy_ref, by_ref, wa_ref, ba_ref,
                 wv_ref, bv_ref,
                 Wv_ref, Hv_ref, sv_ref, hv_ref, g_ref,
                 idx_ref, w_ref, tab_ref):
    q = q_ref[...]
    tab_ref[...] = jnp.dot(v_ref[...], wv_ref[...],
                           preferred_element_type=jnp.float32) + bv_ref[...]
    sox = jnp.dot(q, wx_ref[...], preferred_element_type=jnp.float32) + bx_ref[...]
    soy = jnp.dot(q, wy_ref[...], preferred_element_type=jnp.float32) + by_ref[...]
    logits = jnp.dot(q, wa_ref[...], preferred_element_type=jnp.float32) + ba_ref[...]
    m = jnp.max(logits, axis=1, keepdims=True)
    e = jnp.exp(logits - m)
    s = lax.dot_general(e, g_ref[...], (((1,), (0,)), ((), ())),
                        precision=lax.Precision.HIGHEST)
    aw = e / s
    Wv = Wv_ref[...]
    Hv = Hv_ref[...]
    gx = 2.0 * (rx_ref[...] + sox / Wv) - 1.0
    gy = 2.0 * (ry_ref[...] + soy / Hv) - 1.0
    x = ((gx + 1.0) * Wv - 1.0) * 0.5
    y = ((gy + 1.0) * Hv - 1.0) * 0.5
    x0f = jnp.floor(x)
    y0f = jnp.floor(y)
    fx = x - x0f
    fy = y - y0f
    x0 = x0f.astype(jnp.int32)
    y0 = y0f.astype(jnp.int32)
    Wi = Wv.astype(jnp.int32)
    Hi = Hv.astype(jnp.int32)
    sv = sv_ref[...]
    hv = hv_ref[...]
    boff = boff_ref[...]
    for c, (cy, cx) in enumerate(((0, 0), (0, 1), (1, 0), (1, 1))):
        xc = x0 + cx
        yc = y0 + cy
        wgt = aw * (fx if cx else 1.0 - fx) * (fy if cy else 1.0 - fy)
        valid = (xc >= 0) & (xc < Wi) & (yc >= 0) & (yc < Hi)
        wgt = jnp.where(valid, wgt, 0.0)
        pos = jnp.where(valid, sv + yc * Wi + xc, 0)
        idx_ref[:, pl.ds(c * 128, 128)] = boff + pos * HEADS + hv
        w_ref[:, pl.ds(c * 128, 128)] = wgt


def _tc_out_body(x_ref, w_ref, b_ref, o_ref):
    o_ref[...] = jnp.dot(x_ref[...], w_ref[...],
                         preferred_element_type=jnp.float32) + b_ref[...]


def _sc_body(tab_hbm, idx_hbm, w_hbm, out_hbm, idx_v, w_v, rows_v, out_v, sem):
    wid = lax.axis_index("s") * 2 + lax.axis_index("c")
    base = wid * RPW

    def chunk(g, carry):
        r0 = base + g * CH
        pltpu.sync_copy(idx_hbm.at[pl.ds(r0, CH)], idx_v)
        pltpu.sync_copy(w_hbm.at[pl.ds(r0, CH)], w_v)
        copies = []
        for r in range(CH):
            for c in range(4):
                copies.append(pltpu.async_copy(
                    tab_hbm.at[idx_v.at[r, pl.ds(c * 128, 128)]],
                    rows_v.at[r, pl.ds(c * 128, 128)], sem))
        for cp in copies:
            cp.wait()
        for r in range(CH):
            def hbody(h, carry2):
                z = jnp.zeros((16,), jnp.float32)
                a0 = z
                a1 = z
                j0 = h * LP
                for c in range(4):
                    wv16 = w_v[r, pl.ds(c * 128 + j0, LP)]
                    for k in range(LP):
                        wsc = wv16[k]
                        j = c * 128 + j0 + k
                        a0 = a0 + wsc * rows_v[r, j, pl.ds(0, 16)]
                        a1 = a1 + wsc * rows_v[r, j, pl.ds(16, 16)]
                out_v[r, pl.ds(h * HD, 16)] = a0
                out_v[r, pl.ds(h * HD + 16, 16)] = a1
                return carry2
            lax.fori_loop(0, HEADS, hbody, 0)
        pltpu.sync_copy(out_v, out_hbm.at[pl.ds(r0, CH)])
        return carry

    lax.fori_loop(0, RPW // CH, chunk, 0)


def kernel(query, value, reference_points, spatial_shapes, level_start_index,
           W_samp, b_samp, W_attn, b_attn, W_val, b_val, W_out, b_out):
    q = query.reshape(ROWS, EMBED)
    v = value.reshape(ROWS, EMBED)

    lane = jnp.arange(128, dtype=jnp.int32)
    lvl = (lane // POINTS) % LEVELS
    ssf = spatial_shapes.astype(jnp.float32)
    Wv = ssf[:, 1][lvl][None, :]
    Hv = ssf[:, 0][lvl][None, :]
    sv = level_start_index[lvl][None, :].astype(jnp.int32)
    hv = (lane // LP)[None, :]
    G = (jnp.arange(128)[:, None] // LP ==
         jnp.arange(128)[None, :] // LP).astype(jnp.float32)
    rx_b = reference_points[..., 0][:, :, lvl].reshape(ROWS, 128)
    ry_b = reference_points[..., 1][:, :, lvl].reshape(ROWS, 128)
    boff = (jnp.arange(ROWS, dtype=jnp.int32)[:, None] // NQ) * (NQ * HEADS)
    W_x = W_samp[:, 0::2]
    W_y = W_samp[:, 1::2]
    b_x = b_samp[0::2][None, :]
    b_y = b_samp[1::2][None, :]
    ba = b_attn[None, :]
    bv = b_val[None, :]
    bo = b_out[None, :]

    nblk = ROWS // BLK
    row_spec = lambda c: pl.BlockSpec((BLK, c), lambda i: (i, 0))
    full_spec = lambda r, c: pl.BlockSpec((r, c), lambda i: (0, 0))

    idx, w, tab = pl.pallas_call(
        _tc_pre_body,
        grid=(nblk,),
        in_specs=[
            row_spec(EMBED),
            row_spec(EMBED),
            row_spec(128),
            row_spec(128),
            row_spec(1),
            full_spec(EMBED, 128),
            full_spec(1, 128),
            full_spec(EMBED, 128),
            full_spec(1, 128),
            full_spec(EMBED, 128),
            full_spec(1, 128),
            full_spec(EMBED, EMBED),
            full_spec(1, EMBED),
            full_spec(1, 128),
            full_spec(1, 128),
            full_spec(1, 128),
            full_spec(1, 128),
            full_spec(128, 128),
        ],
        out_specs=[
            pl.BlockSpec((BLK, 512), lambda i: (i, 0)),
            pl.BlockSpec((BLK, 512), lambda i: (i, 0)),
            pl.BlockSpec((BLK, EMBED), lambda i: (i, 0)),
        ],
        out_shape=[
            jax.ShapeDtypeStruct((ROWS, 512), jnp.int32),
            jax.ShapeDtypeStruct((ROWS, 512), jnp.float32),
            jax.ShapeDtypeStruct((ROWS, EMBED), jnp.float32),
        ],
    )(q, v, rx_b, ry_b, boff, W_x, b_x, W_y, b_y, W_attn, ba, W_val, bv,
      Wv, Hv, sv, hv, G)

    table = tab.reshape(ROWS * HEADS, HD)

    sc_call = functools.partial(
        pl.kernel,
        out_type=jax.ShapeDtypeStruct((ROWS, EMBED), jnp.float32),
        mesh=plsc.VectorSubcoreMesh(core_axis_name="c", subcore_axis_name="s"),
        scratch_types=[
            pltpu.VMEM((CH, 512), jnp.int32),
            pltpu.VMEM((CH, 512), jnp.float32),
            pltpu.VMEM((CH, 512, HD), jnp.float32),
            pltpu.VMEM((CH, EMBED), jnp.float32),
            pltpu.SemaphoreType.DMA,
        ],
        compiler_params=pltpu.CompilerParams(use_tc_tiling_on_sc=False),
    )(_sc_body)
    msda = sc_call(table, idx, w)

    out = pl.pallas_call(
        _tc_out_body,
        grid=(nblk,),
        in_specs=[
            row_spec(EMBED),
            full_spec(EMBED, EMBED),
            full_spec(1, EMBED),
        ],
        out_specs=pl.BlockSpec((BLK, EMBED), lambda i: (i, 0)),
        out_shape=jax.ShapeDtypeStruct((ROWS, EMBED), jnp.float32),
    )(msda, W_out, bo)

    return out.reshape(BS, NQ, EMBED)
